# Initial kernel scaffold; baseline (speedup 1.0000x reference)
import functools

import jax
import jax.numpy as jnp
from jax import lax
from jax.experimental import pallas as pl
from jax.experimental.pallas import tpu as pltpu
from jax.experimental.pallas import tpu_sc as plsc

N = 10000
E = 320000
NC = 2
NS = 16
NW = NC * NS
EPW = E // NW
CH = 125
NCHUNK = EPW // CH
RPT = N // NS
DEGW = 16

_mesh = plsc.VectorSubcoreMesh(
    core_axis_name="c", subcore_axis_name="s", num_cores=NC, num_subcores=NS)


def _sc_degree(dst3, zeros_deg, ones_deg):
  @functools.partial(
      pl.kernel,
      out_type=jax.ShapeDtypeStruct((NC * N, DEGW), jnp.float32),
      mesh=_mesh,
      scratch_types=[
          pltpu.VMEM((NCHUNK, CH), jnp.int32),
          pltpu.VMEM((CH, DEGW), jnp.float32),
          pltpu.VMEM_SHARED((N, DEGW), jnp.float32),
      ],
  )
  def deg_kernel(dst_hbm, zeros_hbm, ones_hbm, out_hbm, dst_v, ones_v, acc_sh):
    cid = lax.axis_index("c")
    sid = lax.axis_index("s")
    wid = cid * NS + sid
    pltpu.sync_copy(zeros_hbm, acc_sh.at[pl.ds(sid * RPT, RPT)])
    pltpu.sync_copy(ones_hbm, ones_v)
    pltpu.sync_copy(dst_hbm.at[wid], dst_v)
    plsc.subcore_barrier()

    def body(j, carry):
      pltpu.sync_copy(ones_v, acc_sh.at[dst_v.at[j]], add=True)
      return carry

    lax.fori_loop(0, NCHUNK, body, 0)
    plsc.subcore_barrier()
    pltpu.sync_copy(acc_sh.at[pl.ds(sid * RPT, RPT)],
                    out_hbm.at[pl.ds(cid * N + sid * RPT, RPT)])

  return deg_kernel(dst3, zeros_deg, ones_deg)


def _sc_aggregate(hs, src3, dst3, zeros_f, feat):
  @functools.partial(
      pl.kernel,
      out_type=jax.ShapeDtypeStruct((NC * N, feat), jnp.float32),
      mesh=_mesh,
      scratch_types=[
          pltpu.VMEM((NCHUNK, CH), jnp.int32),
          pltpu.VMEM((NCHUNK, CH), jnp.int32),
          pltpu.VMEM((CH, feat), jnp.float32),
          pltpu.VMEM((CH, feat), jnp.float32),
          pltpu.VMEM_SHARED((N, feat), jnp.float32),
          pltpu.SemaphoreType.DMA,
          pltpu.SemaphoreType.DMA,
      ],
  )
  def agg_kernel(hs_hbm, src_hbm, dst_hbm, zeros_hbm, out_hbm,
                 src_v, dst_v, buf0, buf1, acc_sh, sem0, sem1):
    cid = lax.axis_index("c")
    sid = lax.axis_index("s")
    wid = cid * NS + sid
    pltpu.sync_copy(zeros_hbm, acc_sh.at[pl.ds(sid * RPT, RPT)])
    pltpu.sync_copy(src_hbm.at[wid], src_v)
    pltpu.sync_copy(dst_hbm.at[wid], dst_v)
    plsc.subcore_barrier()

    pltpu.async_copy(hs_hbm.at[src_v.at[0]], buf0, sem0)

    def body(g, carry):
      j0 = 2 * g
      j1 = 2 * g + 1
      pltpu.async_copy(hs_hbm.at[src_v.at[j1]], buf1, sem1)
      pltpu.make_async_copy(hs_hbm.at[src_v.at[j0]], buf0, sem0).wait()
      pltpu.sync_copy(buf0, acc_sh.at[dst_v.at[j0]], add=True)

      @pl.when(g + 1 < NCHUNK // 2)
      def _():
        pltpu.async_copy(hs_hbm.at[src_v.at[j0 + 2]], buf0, sem0)

      pltpu.make_async_copy(hs_hbm.at[src_v.at[j1]], buf1, sem1).wait()
      pltpu.sync_copy(buf1, acc_sh.at[dst_v.at[j1]], add=True)
      return carry

    lax.fori_loop(0, NCHUNK // 2, body, 0)
    plsc.subcore_barrier()
    pltpu.sync_copy(acc_sh.at[pl.ds(sid * RPT, RPT)],
                    out_hbm.at[pl.ds(cid * N + sid * RPT, RPT)])

  return agg_kernel(hs, src3, dst3, zeros_f)


ROWS_B = 1000


def _dinv_block(degp_ref):
  deg = degp_ref[0, :, :1] + degp_ref[1, :, :1] + 1.0
  return lax.rsqrt(deg)


def _tc_prep_body(x_ref, w_ref, degp_ref, o_ref):
  h = jnp.dot(x_ref[...], w_ref[...], preferred_element_type=jnp.float32)
  o_ref[...] = h * _dinv_block(degp_ref)


def _tc_mid_body(a_ref, hs_ref, degp_ref, b_ref, w_ref, o_ref):
  dinv = _dinv_block(degp_ref)
  z = dinv * (a_ref[0] + a_ref[1] + hs_ref[...]) + b_ref[...]
  z = jnp.maximum(z, 0.0)
  o_ref[...] = jnp.dot(z, w_ref[...],
                       preferred_element_type=jnp.float32) * dinv


def _tc_final_body(a_ref, hs_ref, degp_ref, b_ref, o_ref):
  dinv = _dinv_block(degp_ref)
  o = dinv * (a_ref[0] + a_ref[1] + hs_ref[...]) + b_ref[...]
  m = jnp.max(o, axis=1, keepdims=True)
  s = jnp.sum(jnp.exp(o - m), axis=1, keepdims=True)
  o_ref[...] = o - m - jnp.log(s)


def _row_spec(f):
  return pl.BlockSpec((ROWS_B, f), lambda i: (i, 0))


def _part_spec(f):
  return pl.BlockSpec((2, ROWS_B, f), lambda i: (0, i, 0))


_degp_spec = pl.BlockSpec((2, ROWS_B, DEGW), lambda i: (0, i, 0))


def _full_spec(r, c):
  return pl.BlockSpec((r, c), lambda i: (0, 0))


def kernel(x, edge_index, W1, b1, W2, b2):
  src3 = edge_index[0].reshape(NW, NCHUNK, CH)
  dst3 = edge_index[1].reshape(NW, NCHUNK, CH)
  zeros_deg = jnp.zeros((RPT, DEGW), jnp.float32)
  ones_deg = jnp.ones((CH, DEGW), jnp.float32)
  zeros128 = jnp.zeros((RPT, 128), jnp.float32)
  zeros64 = jnp.zeros((RPT, 64), jnp.float32)

  degp = _sc_degree(dst3, zeros_deg, ones_deg).reshape(2, N, DEGW)

  hs1 = pl.pallas_call(
      _tc_prep_body,
      grid=(N // ROWS_B,),
      in_specs=[_row_spec(128), _full_spec(128, 128), _degp_spec],
      out_specs=_row_spec(128),
      out_shape=jax.ShapeDtypeStruct((N, 128), jnp.float32),
  )(x, W1, degp)

  A1 = _sc_aggregate(hs1, src3, dst3, zeros128, 128).reshape(2, N, 128)

  hs2 = pl.pallas_call(
      _tc_mid_body,
      grid=(N // ROWS_B,),
      in_specs=[_part_spec(128), _row_spec(128), _degp_spec,
                _full_spec(1, 128), _full_spec(128, 64)],
      out_specs=_row_spec(64),
      out_shape=jax.ShapeDtypeStruct((N, 64), jnp.float32),
  )(A1, hs1, degp, b1.reshape(1, 128), W2)

  A2 = _sc_aggregate(hs2, src3, dst3, zeros64, 64).reshape(2, N, 64)

  out = pl.pallas_call(
      _tc_final_body,
      grid=(N // ROWS_B,),
      in_specs=[_part_spec(64), _row_spec(64), _degp_spec,
                _full_spec(1, 64)],
      out_specs=_row_spec(64),
      out_shape=jax.ShapeDtypeStruct((N, 64), jnp.float32),
  )(A2, hs2, degp, b2.reshape(1, 64))

  return out


# speedup vs baseline: 24.7451x; 24.7451x over previous
import functools

import jax
import jax.numpy as jnp
from jax import lax
from jax.experimental import pallas as pl
from jax.experimental.pallas import tpu as pltpu
from jax.experimental.pallas import tpu_sc as plsc

N = 10000
E = 320000
NC = 2
NS = 16
NW = NC * NS
EPW = E // NW
CH = 125
NCHUNK = EPW // CH
IB = 8
NBLK = NCHUNK // IB
NPAD = 10240
RPT = NPAD // NS
DEGW = 128

_mesh = plsc.VectorSubcoreMesh(
    core_axis_name="c", subcore_axis_name="s", num_cores=NC, num_subcores=NS)


def _sc_degree(dst3, zeros_deg, ones_deg):
  @functools.partial(
      pl.kernel,
      out_type=jax.ShapeDtypeStruct((NC * NPAD, DEGW), jnp.float32),
      mesh=_mesh,
      scratch_types=[
          pltpu.VMEM((IB, CH), jnp.int32),
          pltpu.VMEM((CH, DEGW), jnp.float32),
          pltpu.VMEM_SHARED((NPAD, DEGW), jnp.float32),
      ],
  )
  def deg_kernel(dst_hbm, zeros_hbm, ones_hbm, out_hbm, dst_v, ones_v, acc_sh):
    cid = lax.axis_index("c")
    sid = lax.axis_index("s")
    wid = cid * NS + sid
    pltpu.sync_copy(zeros_hbm, acc_sh.at[pl.ds(sid * RPT, RPT)])
    pltpu.sync_copy(ones_hbm, ones_v)
    plsc.subcore_barrier()

    def block_body(ib, carry):
      off = pl.multiple_of(ib * IB, IB)
      pltpu.sync_copy(dst_hbm.at[wid, pl.ds(off, IB)], dst_v)
      for j in range(IB):
        pltpu.sync_copy(ones_v, acc_sh.at[dst_v.at[j]], add=True)
      return carry

    lax.fori_loop(0, NBLK, block_body, 0)
    plsc.subcore_barrier()
    pltpu.sync_copy(acc_sh.at[pl.ds(sid * RPT, RPT)],
                    out_hbm.at[pl.ds(cid * NPAD + sid * RPT, RPT)])

  return deg_kernel(dst3, zeros_deg, ones_deg)


def _sc_aggregate(hs, src3, dst3, zeros_f, feat):
  @functools.partial(
      pl.kernel,
      out_type=jax.ShapeDtypeStruct((NC * NPAD, feat), jnp.float32),
      mesh=_mesh,
      scratch_types=[
          pltpu.VMEM((IB, CH), jnp.int32),
          pltpu.VMEM((IB, CH), jnp.int32),
          pltpu.VMEM((CH, feat), jnp.float32),
          pltpu.VMEM((CH, feat), jnp.float32),
          pltpu.VMEM_SHARED((NPAD, feat), jnp.float32),
          pltpu.SemaphoreType.DMA,
          pltpu.SemaphoreType.DMA,
      ],
  )
  def agg_kernel(hs_hbm, src_hbm, dst_hbm, zeros_hbm, out_hbm,
                 src_v, dst_v, buf0, buf1, acc_sh, sem0, sem1):
    cid = lax.axis_index("c")
    sid = lax.axis_index("s")
    wid = cid * NS + sid
    pltpu.sync_copy(zeros_hbm, acc_sh.at[pl.ds(sid * RPT, RPT)])
    plsc.subcore_barrier()
    bufs = (buf0, buf1)
    sems = (sem0, sem1)

    def block_body(ib, carry):
      off = pl.multiple_of(ib * IB, IB)
      pltpu.sync_copy(src_hbm.at[wid, pl.ds(off, IB)], src_v)
      pltpu.sync_copy(dst_hbm.at[wid, pl.ds(off, IB)], dst_v)
      pltpu.async_copy(hs_hbm.at[src_v.at[0]], buf0, sem0)
      for j in range(IB):
        cb = j % 2
        nb = (j + 1) % 2
        if j + 1 < IB:
          pltpu.async_copy(hs_hbm.at[src_v.at[j + 1]], bufs[nb], sems[nb])
        pltpu.make_async_copy(hs_hbm.at[src_v.at[j]], bufs[cb], sems[cb]).wait()
        pltpu.sync_copy(bufs[cb], acc_sh.at[dst_v.at[j]], add=True)
      return carry

    lax.fori_loop(0, NBLK, block_body, 0)
    plsc.subcore_barrier()
    pltpu.sync_copy(acc_sh.at[pl.ds(sid * RPT, RPT)],
                    out_hbm.at[pl.ds(cid * NPAD + sid * RPT, RPT)])

  return agg_kernel(hs, src3, dst3, zeros_f)


ROWS_B = 1000


def _dinv_block(degp_ref):
  deg = degp_ref[0, :, :1] + degp_ref[1, :, :1] + 1.0
  return lax.rsqrt(deg)


def _tc_prep_body(x_ref, w_ref, degp_ref, o_ref):
  h = jnp.dot(x_ref[...], w_ref[...], preferred_element_type=jnp.float32)
  o_ref[...] = h * _dinv_block(degp_ref)


def _tc_mid_body(a_ref, hs_ref, degp_ref, b_ref, w_ref, o_ref):
  dinv = _dinv_block(degp_ref)
  z = dinv * (a_ref[0] + a_ref[1] + hs_ref[...]) + b_ref[...]
  z = jnp.maximum(z, 0.0)
  o_ref[...] = jnp.dot(z, w_ref[...],
                       preferred_element_type=jnp.float32) * dinv


def _tc_final_body(a_ref, hs_ref, degp_ref, b_ref, o_ref):
  dinv = _dinv_block(degp_ref)
  o = dinv * (a_ref[0, :, :64] + a_ref[1, :, :64] + hs_ref[:, :64]) + b_ref[...]
  m = jnp.max(o, axis=1, keepdims=True)
  s = jnp.sum(jnp.exp(o - m), axis=1, keepdims=True)
  o_ref[...] = o - m - jnp.log(s)


def _row_spec(f):
  return pl.BlockSpec((ROWS_B, f), lambda i: (i, 0))


def _part_spec(f):
  return pl.BlockSpec((2, ROWS_B, f), lambda i: (0, i, 0))


_degp_spec = pl.BlockSpec((2, ROWS_B, DEGW), lambda i: (0, i, 0))


def _full_spec(r, c):
  return pl.BlockSpec((r, c), lambda i: (0, 0))


def kernel(x, edge_index, W1, b1, W2, b2):
  src3 = edge_index[0].reshape(NW, NCHUNK, CH)
  dst3 = edge_index[1].reshape(NW, NCHUNK, CH)
  zeros_deg = jnp.zeros((RPT, DEGW), jnp.float32)
  ones_deg = jnp.ones((CH, DEGW), jnp.float32)
  zeros128 = jnp.zeros((RPT, 128), jnp.float32)

  degp = _sc_degree(dst3, zeros_deg, ones_deg).reshape(2, NPAD, DEGW)

  hs1 = pl.pallas_call(
      _tc_prep_body,
      grid=(N // ROWS_B,),
      in_specs=[_row_spec(128), _full_spec(128, 128), _degp_spec],
      out_specs=_row_spec(128),
      out_shape=jax.ShapeDtypeStruct((N, 128), jnp.float32),
  )(x, W1, degp)

  A1 = _sc_aggregate(hs1, src3, dst3, zeros128, 128).reshape(2, NPAD, 128)

  W2p = jnp.concatenate([W2, jnp.zeros((128, 64), jnp.float32)], axis=1)

  hs2 = pl.pallas_call(
      _tc_mid_body,
      grid=(N // ROWS_B,),
      in_specs=[_part_spec(128), _row_spec(128), _degp_spec,
                _full_spec(1, 128), _full_spec(128, 128)],
      out_specs=_row_spec(128),
      out_shape=jax.ShapeDtypeStruct((N, 128), jnp.float32),
  )(A1, hs1, degp, b1.reshape(1, 128), W2p)

  A2 = _sc_aggregate(hs2, src3, dst3, zeros128, 128).reshape(2, NPAD, 128)

  out = pl.pallas_call(
      _tc_final_body,
      grid=(N // ROWS_B,),
      in_specs=[_part_spec(128), _row_spec(128), _degp_spec,
                _full_spec(1, 64)],
      out_specs=_row_spec(64),
      out_shape=jax.ShapeDtypeStruct((N, 64), jnp.float32),
  )(A2, hs2, degp, b2.reshape(1, 64))

  return out


# speedup vs baseline: 27.4144x; 1.1079x over previous
import functools

import jax
import jax.numpy as jnp
from jax import lax
from jax.experimental import pallas as pl
from jax.experimental.pallas import tpu as pltpu
from jax.experimental.pallas import tpu_sc as plsc

N = 10000
E = 320000
NC = 2
NS = 16
NW = NC * NS
EPW = E // NW
CH = 100
NCHUNK = EPW // CH
IB = 20
NBLK = NCHUNK // IB
NPAD = 10112
RPT = NPAD // NS
DEGW = 128

_mesh = plsc.VectorSubcoreMesh(
    core_axis_name="c", subcore_axis_name="s", num_cores=NC, num_subcores=NS)


def _sc_degree(dst3, zeros_deg, ones_deg):
  @functools.partial(
      pl.kernel,
      out_type=jax.ShapeDtypeStruct((NC * NPAD, DEGW), jnp.float32),
      mesh=_mesh,
      scratch_types=[
          pltpu.VMEM((IB, CH), jnp.int32),
          pltpu.VMEM((CH, DEGW), jnp.float32),
          pltpu.VMEM_SHARED((NPAD, DEGW), jnp.float32),
      ],
  )
  def deg_kernel(dst_hbm, zeros_hbm, ones_hbm, out_hbm, dst_v, ones_v, acc_sh):
    cid = lax.axis_index("c")
    sid = lax.axis_index("s")
    wid = cid * NS + sid
    pltpu.sync_copy(zeros_hbm, acc_sh.at[pl.ds(sid * RPT, RPT)])
    pltpu.sync_copy(ones_hbm, ones_v)
    plsc.subcore_barrier()

    def block_body(ib, carry):
      pltpu.sync_copy(dst_hbm.at[wid, ib], dst_v)
      for j in range(IB):
        pltpu.sync_copy(ones_v, acc_sh.at[dst_v.at[j]], add=True)
      return carry

    lax.fori_loop(0, NBLK, block_body, 0)
    plsc.subcore_barrier()
    pltpu.sync_copy(acc_sh.at[pl.ds(sid * RPT, RPT)],
                    out_hbm.at[pl.ds(cid * NPAD + sid * RPT, RPT)])

  return deg_kernel(dst3, zeros_deg, ones_deg)


def _sc_aggregate(hs, src3, dst3, zeros_f, feat):
  @functools.partial(
      pl.kernel,
      out_type=jax.ShapeDtypeStruct((NC * NPAD, feat), jnp.float32),
      mesh=_mesh,
      scratch_types=[
          pltpu.VMEM((IB, CH), jnp.int32),
          pltpu.VMEM((IB, CH), jnp.int32),
          pltpu.VMEM((CH, feat), jnp.float32),
          pltpu.VMEM((CH, feat), jnp.float32),
          pltpu.VMEM((CH, feat), jnp.float32),
          pltpu.VMEM_SHARED((NPAD, feat), jnp.float32),
          pltpu.SemaphoreType.DMA,
          pltpu.SemaphoreType.DMA,
          pltpu.SemaphoreType.DMA,
          pltpu.SemaphoreType.DMA,
          pltpu.SemaphoreType.DMA,
          pltpu.SemaphoreType.DMA,
      ],
  )
  def agg_kernel(hs_hbm, src_hbm, dst_hbm, zeros_hbm, out_hbm,
                 src_v, dst_v, buf0, buf1, buf2, acc_sh,
                 g0, g1, g2, s0, s1, s2):
    cid = lax.axis_index("c")
    sid = lax.axis_index("s")
    wid = cid * NS + sid
    pltpu.sync_copy(zeros_hbm, acc_sh.at[pl.ds(sid * RPT, RPT)])
    plsc.subcore_barrier()
    bufs = (buf0, buf1, buf2)
    gsems = (g0, g1, g2)
    ssems = (s0, s1, s2)

    def gather(j, b):
      return pltpu.async_copy(hs_hbm.at[src_v.at[j]], bufs[b], gsems[b])

    def scatter(j, b):
      return pltpu.async_copy(bufs[b], acc_sh.at[dst_v.at[j]], ssems[b],
                              add=True)

    def block_body(ib, carry):
      pltpu.sync_copy(src_hbm.at[wid, ib], src_v)
      pltpu.sync_copy(dst_hbm.at[wid, ib], dst_v)
      gd = {0: gather(0, 0), 1: gather(1, 1)}
      sd = {}
      for jj in range(IB):
        b = jj % 3
        gd[b].wait()
        sd[b] = scatter(jj, b)
        nxt = jj + 2
        if nxt < IB:
          nb = nxt % 3
          if jj >= 1:
            sd[nb].wait()
          gd[nb] = gather(nxt, nb)
      sd[(IB - 3) % 3].wait()
      sd[(IB - 2) % 3].wait()
      sd[(IB - 1) % 3].wait()
      return carry

    lax.fori_loop(0, NBLK, block_body, 0)
    plsc.subcore_barrier()
    pltpu.sync_copy(acc_sh.at[pl.ds(sid * RPT, RPT)],
                    out_hbm.at[pl.ds(cid * NPAD + sid * RPT, RPT)])

  return agg_kernel(hs, src3, dst3, zeros_f)


ROWS_B = 1000


def _dinv_block(degp_ref):
  deg = degp_ref[0, :, :1] + degp_ref[1, :, :1] + 1.0
  return lax.rsqrt(deg)


def _tc_prep_body(x_ref, w_ref, degp_ref, o_ref):
  h = jnp.dot(x_ref[...], w_ref[...], preferred_element_type=jnp.float32)
  o_ref[...] = h * _dinv_block(degp_ref)


def _tc_mid_body(a_ref, hs_ref, degp_ref, b_ref, w_ref, o_ref):
  dinv = _dinv_block(degp_ref)
  z = dinv * (a_ref[0] + a_ref[1] + hs_ref[...]) + b_ref[...]
  z = jnp.maximum(z, 0.0)
  o_ref[...] = jnp.dot(z, w_ref[...],
                       preferred_element_type=jnp.float32) * dinv


def _tc_final_body(a_ref, hs_ref, degp_ref, b_ref, o_ref):
  dinv = _dinv_block(degp_ref)
  o = dinv * (a_ref[0, :, :64] + a_ref[1, :, :64] + hs_ref[:, :64]) + b_ref[...]
  m = jnp.max(o, axis=1, keepdims=True)
  s = jnp.sum(jnp.exp(o - m), axis=1, keepdims=True)
  o_ref[...] = o - m - jnp.log(s)


def _row_spec(f):
  return pl.BlockSpec((ROWS_B, f), lambda i: (i, 0))


def _part_spec(f):
  return pl.BlockSpec((2, ROWS_B, f), lambda i: (0, i, 0))


_degp_spec = pl.BlockSpec((2, ROWS_B, DEGW), lambda i: (0, i, 0))


def _full_spec(r, c):
  return pl.BlockSpec((r, c), lambda i: (0, 0))


def kernel(x, edge_index, W1, b1, W2, b2):
  src3 = edge_index[0].reshape(NW, NBLK, IB, CH)
  dst3 = edge_index[1].reshape(NW, NBLK, IB, CH)
  zeros_deg = jnp.zeros((RPT, DEGW), jnp.float32)
  ones_deg = jnp.ones((CH, DEGW), jnp.float32)
  zeros128 = jnp.zeros((RPT, 128), jnp.float32)

  degp = _sc_degree(dst3, zeros_deg, ones_deg).reshape(2, NPAD, DEGW)

  hs1 = pl.pallas_call(
      _tc_prep_body,
      grid=(N // ROWS_B,),
      in_specs=[_row_spec(128), _full_spec(128, 128), _degp_spec],
      out_specs=_row_spec(128),
      out_shape=jax.ShapeDtypeStruct((N, 128), jnp.float32),
  )(x, W1, degp)

  A1 = _sc_aggregate(hs1, src3, dst3, zeros128, 128).reshape(2, NPAD, 128)

  W2p = jnp.concatenate([W2, jnp.zeros((128, 64), jnp.float32)], axis=1)

  hs2 = pl.pallas_call(
      _tc_mid_body,
      grid=(N // ROWS_B,),
      in_specs=[_part_spec(128), _row_spec(128), _degp_spec,
                _full_spec(1, 128), _full_spec(128, 128)],
      out_specs=_row_spec(128),
      out_shape=jax.ShapeDtypeStruct((N, 128), jnp.float32),
  )(A1, hs1, degp, b1.reshape(1, 128), W2p)

  A2 = _sc_aggregate(hs2, src3, dst3, zeros128, 128).reshape(2, NPAD, 128)

  out = pl.pallas_call(
      _tc_final_body,
      grid=(N // ROWS_B,),
      in_specs=[_part_spec(128), _row_spec(128), _degp_spec,
                _full_spec(1, 64)],
      out_specs=_row_spec(64),
      out_shape=jax.ShapeDtypeStruct((N, 64), jnp.float32),
  )(A2, hs2, degp, b2.reshape(1, 64))

  return out
